# Initial kernel scaffold; baseline (speedup 1.0000x reference)
import jax
import jax.numpy as jnp
from jax.experimental import pallas as pl


def kernel(x_seq, a_seq, feat_seq, w1, b1, w2, b2, fcw_flat, fcb, wii, wgi, whi, bi, wif, wgf, whf, bf, wig, wgg, whg, bg, wio, wgo, who, bo):
    raise NotImplementedError("write your pallas kernel here")


# speedup vs baseline: 1.5352x; 1.5352x over previous
import functools

import jax
import jax.numpy as jnp
from jax.experimental import pallas as pl
from jax.experimental.pallas import tpu as pltpu


def _gnn_phase(a_ref, feat_ref, w1_ref, b1_ref, w2_ref, b2_ref,
               fcw_ref, fcb_ref, wg_ref, b4_ref, zg_ref, *, n_nodes, gh):
    f32 = jnp.float32
    N, GH = n_nodes, gh
    NG = N * GH
    a = a_ref[0]
    feat = feat_ref[0]

    xw1 = jnp.dot(feat, w1_ref[...], preferred_element_type=f32)
    h1 = jnp.maximum(jnp.dot(a, xw1, preferred_element_type=f32)
                     + b1_ref[...], 0.0)
    h2 = jnp.dot(a, jnp.dot(h1, w2_ref[...], preferred_element_type=f32),
                 preferred_element_type=f32) + b2_ref[...]

    col = jax.lax.broadcasted_iota(jnp.int32, (N, NG), 1)
    row = jax.lax.broadcasted_iota(jnp.int32, (N, NG), 0)
    mask = (col // GH == row).astype(f32)
    kcol = jax.lax.broadcasted_iota(jnp.int32, (GH, NG), 1)
    krow = jax.lax.broadcasted_iota(jnp.int32, (GH, NG), 0)
    ktile = (kcol % GH == krow).astype(f32)
    tiled = jnp.dot(h2, ktile, preferred_element_type=f32) * mask
    flat = jnp.dot(jnp.ones((1, N), f32), tiled,
                   preferred_element_type=f32)

    gnn = jnp.maximum(jnp.dot(flat, fcw_ref[...], preferred_element_type=f32)
                      + fcb_ref[...], 0.0)
    zg_ref[0] = (jnp.dot(gnn, wg_ref[...], preferred_element_type=f32)
                 + b4_ref[...])


def _lstm_phase(x_ref, zg_ref, wx_ref, wh_ref, out_ref, c_ref,
                *, seq_len, hidden, bblk, in_dim):
    f32 = jnp.float32
    T, H, B = seq_len, hidden, bblk
    x = x_ref[...].reshape(T * B, in_dim)
    zx = jnp.dot(x, wx_ref[...], preferred_element_type=f32)
    wh = wh_ref[...]

    h = jnp.zeros((B, H), f32)
    c = jnp.zeros((B, H), f32)
    outs = []
    for t in range(T):
        z = (zx[t * B:(t + 1) * B, :] + zg_ref[t]
             + jnp.dot(h, wh, preferred_element_type=f32))
        s = jax.nn.sigmoid(z[:, :3 * H])
        g_t = jnp.tanh(z[:, 3 * H:])
        c = s[:, H:2 * H] * c + s[:, :H] * g_t
        h = s[:, 2 * H:] * jnp.tanh(c)
        outs.append(h)
    out_ref[...] = jnp.stack(outs, axis=0)
    c_ref[...] = c


def kernel(x_seq, a_seq, feat_seq, w1, b1, w2, b2, fcw_flat, fcb,
           wii, wgi, whi, bi, wif, wgf, whf, bf,
           wig, wgg, whg, bg, wio, wgo, who, bo):
    T, B, I = x_seq.shape
    _, N, F = feat_seq.shape
    GH = w2.shape[0]
    H = whi.shape[0]
    f32 = jnp.float32

    wx = jnp.concatenate([wii, wif, wio, wig], axis=1)
    wg = jnp.concatenate([wgi, wgf, wgo, wgg], axis=1)
    wh = jnp.concatenate([whi, whf, who, whg], axis=1)
    b4 = jnp.concatenate([bi, bf, bo, bg], axis=1)

    zg = pl.pallas_call(
        functools.partial(_gnn_phase, n_nodes=N, gh=GH),
        out_shape=jax.ShapeDtypeStruct((T, 1, 4 * H), f32),
        grid=(T,),
        in_specs=[
            pl.BlockSpec((1, N, N), lambda t: (t, 0, 0)),
            pl.BlockSpec((1, N, F), lambda t: (t, 0, 0)),
            pl.BlockSpec(w1.shape, lambda t: (0, 0)),
            pl.BlockSpec(b1.shape, lambda t: (0, 0)),
            pl.BlockSpec(w2.shape, lambda t: (0, 0)),
            pl.BlockSpec(b2.shape, lambda t: (0, 0)),
            pl.BlockSpec(fcw_flat.shape, lambda t: (0, 0)),
            pl.BlockSpec(fcb.shape, lambda t: (0, 0)),
            pl.BlockSpec(wg.shape, lambda t: (0, 0)),
            pl.BlockSpec(b4.shape, lambda t: (0, 0)),
        ],
        out_specs=pl.BlockSpec((1, 1, 4 * H), lambda t: (t, 0, 0)),
        compiler_params=pltpu.CompilerParams(
            dimension_semantics=("parallel",)),
    )(a_seq, feat_seq, w1, b1, w2, b2, fcw_flat, fcb, wg, b4)

    nb = 2 if B % 2 == 0 else 1
    BB = B // nb
    out, c_T = pl.pallas_call(
        functools.partial(_lstm_phase, seq_len=T, hidden=H, bblk=BB, in_dim=I),
        out_shape=(
            jax.ShapeDtypeStruct((T, B, H), f32),
            jax.ShapeDtypeStruct((B, H), f32),
        ),
        grid=(nb,),
        in_specs=[
            pl.BlockSpec((T, BB, I), lambda b: (0, b, 0)),
            pl.BlockSpec((T, 1, 4 * H), lambda b: (0, 0, 0)),
            pl.BlockSpec((I, 4 * H), lambda b: (0, 0)),
            pl.BlockSpec((H, 4 * H), lambda b: (0, 0)),
        ],
        out_specs=(
            pl.BlockSpec((T, BB, H), lambda b: (0, b, 0)),
            pl.BlockSpec((BB, H), lambda b: (b, 0)),
        ),
        compiler_params=pltpu.CompilerParams(
            dimension_semantics=("parallel",)),
    )(x_seq, zg, wx, wh)

    return out, out[-1], c_T


# speedup vs baseline: 1.7537x; 1.1423x over previous
import functools

import jax
import jax.numpy as jnp
from jax.experimental import pallas as pl
from jax.experimental.pallas import tpu as pltpu


def _gnn_phase(a_ref, feat_ref, w1_ref, b1_ref, w2_ref, b2_ref,
               fcw_ref, fcb_ref,
               wgi_ref, wgf_ref, wgo_ref, wgg_ref,
               bi_ref, bf_ref, bo_ref, bg_ref,
               zg_ref, *, n_nodes, gh):
    f32 = jnp.float32
    N, GH = n_nodes, gh
    NG = N * GH
    a = a_ref[0]
    feat = feat_ref[0]

    xw1 = jnp.dot(feat, w1_ref[...], preferred_element_type=f32)
    h1 = jnp.maximum(jnp.dot(a, xw1, preferred_element_type=f32)
                     + b1_ref[...], 0.0)
    h2 = jnp.dot(a, jnp.dot(h1, w2_ref[...], preferred_element_type=f32),
                 preferred_element_type=f32) + b2_ref[...]

    col = jax.lax.broadcasted_iota(jnp.int32, (N, NG), 1)
    row = jax.lax.broadcasted_iota(jnp.int32, (N, NG), 0)
    mask = (col // GH == row).astype(f32)
    kcol = jax.lax.broadcasted_iota(jnp.int32, (GH, NG), 1)
    krow = jax.lax.broadcasted_iota(jnp.int32, (GH, NG), 0)
    ktile = (kcol % GH == krow).astype(f32)
    tiled = jnp.dot(h2, ktile, preferred_element_type=f32) * mask
    flat = jnp.dot(jnp.ones((1, N), f32), tiled,
                   preferred_element_type=f32)

    gnn = jnp.maximum(jnp.dot(flat, fcw_ref[...], preferred_element_type=f32)
                      + fcb_ref[...], 0.0)
    wg = jnp.concatenate([wgi_ref[...], wgf_ref[...],
                          wgo_ref[...], wgg_ref[...]], axis=1)
    b4 = jnp.concatenate([bi_ref[...], bf_ref[...],
                          bo_ref[...], bg_ref[...]], axis=1)
    zg_ref[0] = jnp.dot(gnn, wg, preferred_element_type=f32) + b4


def _lstm_phase(x_ref, zg_ref,
                wii_ref, wif_ref, wio_ref, wig_ref,
                whi_ref, whf_ref, who_ref, whg_ref,
                out_ref, c_ref, hN_ref,
                *, seq_len, hidden, bblk, in_dim):
    f32 = jnp.float32
    T, H, B = seq_len, hidden, bblk
    wx = jnp.concatenate([wii_ref[...], wif_ref[...],
                          wio_ref[...], wig_ref[...]], axis=1)
    wh = jnp.concatenate([whi_ref[...], whf_ref[...],
                          who_ref[...], whg_ref[...]], axis=1)
    x = x_ref[...].reshape(T * B, in_dim)
    zx = jnp.dot(x, wx, preferred_element_type=f32)

    h = jnp.zeros((B, H), f32)
    c = jnp.zeros((B, H), f32)
    outs = []
    for t in range(T):
        z = (zx[t * B:(t + 1) * B, :] + zg_ref[t]
             + jnp.dot(h, wh, preferred_element_type=f32))
        s = jax.nn.sigmoid(z[:, :3 * H])
        g_t = jnp.tanh(z[:, 3 * H:])
        c = s[:, H:2 * H] * c + s[:, :H] * g_t
        h = s[:, 2 * H:] * jnp.tanh(c)
        outs.append(h)
    out_ref[...] = jnp.stack(outs, axis=0)
    c_ref[...] = c
    hN_ref[...] = h


def kernel(x_seq, a_seq, feat_seq, w1, b1, w2, b2, fcw_flat, fcb,
           wii, wgi, whi, bi, wif, wgf, whf, bf,
           wig, wgg, whg, bg, wio, wgo, who, bo):
    T, B, I = x_seq.shape
    _, N, F = feat_seq.shape
    GH = w2.shape[0]
    H = whi.shape[0]
    f32 = jnp.float32

    def full(arr):
        return pl.BlockSpec(arr.shape, lambda t, _nd=arr.ndim: (0,) * _nd)

    zg = pl.pallas_call(
        functools.partial(_gnn_phase, n_nodes=N, gh=GH),
        out_shape=jax.ShapeDtypeStruct((T, 1, 4 * H), f32),
        grid=(T,),
        in_specs=[
            pl.BlockSpec((1, N, N), lambda t: (t, 0, 0)),
            pl.BlockSpec((1, N, F), lambda t: (t, 0, 0)),
            full(w1), full(b1), full(w2), full(b2),
            full(fcw_flat), full(fcb),
            full(wgi), full(wgf), full(wgo), full(wgg),
            full(bi), full(bf), full(bo), full(bg),
        ],
        out_specs=pl.BlockSpec((1, 1, 4 * H), lambda t: (t, 0, 0)),
        compiler_params=pltpu.CompilerParams(
            dimension_semantics=("parallel",)),
    )(a_seq, feat_seq, w1, b1, w2, b2, fcw_flat, fcb,
      wgi, wgf, wgo, wgg, bi, bf, bo, bg)

    nb = 2 if B % 2 == 0 else 1
    BB = B // nb
    out, c_T, h_T = pl.pallas_call(
        functools.partial(_lstm_phase, seq_len=T, hidden=H, bblk=BB, in_dim=I),
        out_shape=(
            jax.ShapeDtypeStruct((T, B, H), f32),
            jax.ShapeDtypeStruct((B, H), f32),
            jax.ShapeDtypeStruct((B, H), f32),
        ),
        grid=(nb,),
        in_specs=[
            pl.BlockSpec((T, BB, I), lambda b: (0, b, 0)),
            pl.BlockSpec((T, 1, 4 * H), lambda b: (0, 0, 0)),
            full(wii), full(wif), full(wio), full(wig),
            full(whi), full(whf), full(who), full(whg),
        ],
        out_specs=(
            pl.BlockSpec((T, BB, H), lambda b: (0, b, 0)),
            pl.BlockSpec((BB, H), lambda b: (b, 0)),
            pl.BlockSpec((BB, H), lambda b: (b, 0)),
        ),
        compiler_params=pltpu.CompilerParams(
            dimension_semantics=("parallel",)),
    )(x_seq, zg, wii, wif, wio, wig, whi, whf, who, whg)

    return out, h_T, c_T


# speedup vs baseline: 1.9721x; 1.1245x over previous
import functools

import jax
import jax.numpy as jnp
from jax.experimental import pallas as pl
from jax.experimental.pallas import tpu as pltpu


def _fused_kernel(x_ref, a_ref, feat_ref,
                  w1_ref, b1_ref, w2_ref, b2_ref, fcw_ref, fcb_ref,
                  wgi_ref, wgf_ref, wgo_ref, wgg_ref,
                  bi_ref, bf_ref, bo_ref, bg_ref,
                  wii_ref, wif_ref, wio_ref, wig_ref,
                  whi_ref, whf_ref, who_ref, whg_ref,
                  out_ref, c_ref, hN_ref,
                  zg_s,
                  *, n_nodes, gh, seq_len, batch, hidden, in_dim, t_blk):
    f32 = jnp.float32
    N, GH, T, B, H = n_nodes, gh, seq_len, batch, hidden
    NG = N * GH
    j = pl.program_id(0)
    n_gnn_steps = T // t_blk

    @pl.when(j < n_gnn_steps)
    def _gnn_step():
        feat = feat_ref[...].reshape(t_blk * N, feat_ref.shape[2])
        xw1 = jnp.dot(feat, w1_ref[...], preferred_element_type=f32)

        col = jax.lax.broadcasted_iota(jnp.int32, (N, NG), 1)
        row = jax.lax.broadcasted_iota(jnp.int32, (N, NG), 0)
        mask = (col // GH == row).astype(f32)
        kcol = jax.lax.broadcasted_iota(jnp.int32, (GH, NG), 1)
        krow = jax.lax.broadcasted_iota(jnp.int32, (GH, NG), 0)
        ktile = (kcol % GH == krow).astype(f32)
        ones_row = jnp.ones((1, N), f32)

        h1s = []
        for i in range(t_blk):
            a_i = a_ref[i]
            h1s.append(jnp.maximum(
                jnp.dot(a_i, xw1[i * N:(i + 1) * N, :],
                        preferred_element_type=f32) + b1_ref[...], 0.0))
        h1 = jnp.concatenate(h1s, axis=0)
        hw = jnp.dot(h1, w2_ref[...], preferred_element_type=f32)

        flats = []
        for i in range(t_blk):
            h2_i = (jnp.dot(a_ref[i], hw[i * N:(i + 1) * N, :],
                            preferred_element_type=f32) + b2_ref[...])
            tiled = jnp.dot(h2_i, ktile, preferred_element_type=f32) * mask
            flats.append(jnp.dot(ones_row, tiled,
                                 preferred_element_type=f32))
        flat = jnp.concatenate(flats, axis=0)

        gnn = jnp.maximum(jnp.dot(flat, fcw_ref[...],
                                  preferred_element_type=f32)
                          + fcb_ref[...], 0.0)
        wg = jnp.concatenate([wgi_ref[...], wgf_ref[...],
                              wgo_ref[...], wgg_ref[...]], axis=1)
        b4 = jnp.concatenate([bi_ref[...], bf_ref[...],
                              bo_ref[...], bg_ref[...]], axis=1)
        zg_s[j] = jnp.dot(gnn, wg, preferred_element_type=f32) + b4

    @pl.when(j == n_gnn_steps)
    def _lstm_step():
        wx = jnp.concatenate([wii_ref[...], wif_ref[...],
                              wio_ref[...], wig_ref[...]], axis=1)
        wh = jnp.concatenate([whi_ref[...], whf_ref[...],
                              who_ref[...], whg_ref[...]], axis=1)
        x = x_ref[...].reshape(T * B, in_dim)
        zx = jnp.dot(x, wx, preferred_element_type=f32)

        h = jnp.zeros((B, H), f32)
        c = jnp.zeros((B, H), f32)
        outs = []
        for t in range(T):
            z = (zx[t * B:(t + 1) * B, :]
                 + zg_s[t // t_blk, t % t_blk:t % t_blk + 1, :]
                 + jnp.dot(h, wh, preferred_element_type=f32))
            s = jax.nn.sigmoid(z[:, :3 * H])
            g_t = jnp.tanh(z[:, 3 * H:])
            c = s[:, H:2 * H] * c + s[:, :H] * g_t
            h = s[:, 2 * H:] * jnp.tanh(c)
            outs.append(h)
        out_ref[...] = jnp.stack(outs, axis=0)
        c_ref[...] = c
        hN_ref[...] = h


def kernel(x_seq, a_seq, feat_seq, w1, b1, w2, b2, fcw_flat, fcb,
           wii, wgi, whi, bi, wif, wgf, whf, bf,
           wig, wgg, whg, bg, wio, wgo, who, bo):
    T, B, I = x_seq.shape
    _, N, F = feat_seq.shape
    GH = w2.shape[0]
    H = whi.shape[0]
    f32 = jnp.float32
    T_BLK = 2
    n_gnn = T // T_BLK

    def full(arr):
        return pl.BlockSpec(arr.shape, lambda j, _nd=arr.ndim: (0,) * _nd)

    out, c_T, h_T = pl.pallas_call(
        functools.partial(_fused_kernel, n_nodes=N, gh=GH, seq_len=T,
                          batch=B, hidden=H, in_dim=I, t_blk=T_BLK),
        out_shape=(
            jax.ShapeDtypeStruct((T, B, H), f32),
            jax.ShapeDtypeStruct((B, H), f32),
            jax.ShapeDtypeStruct((B, H), f32),
        ),
        grid=(n_gnn + 1,),
        in_specs=[
            full(x_seq),
            pl.BlockSpec((T_BLK, N, N),
                         lambda j: (jnp.minimum(j, n_gnn - 1), 0, 0)),
            pl.BlockSpec((T_BLK, N, F),
                         lambda j: (jnp.minimum(j, n_gnn - 1), 0, 0)),
            full(w1), full(b1), full(w2), full(b2), full(fcw_flat), full(fcb),
            full(wgi), full(wgf), full(wgo), full(wgg),
            full(bi), full(bf), full(bo), full(bg),
            full(wii), full(wif), full(wio), full(wig),
            full(whi), full(whf), full(who), full(whg),
        ],
        out_specs=(
            pl.BlockSpec((T, B, H), lambda j: (0, 0, 0)),
            pl.BlockSpec((B, H), lambda j: (0, 0)),
            pl.BlockSpec((B, H), lambda j: (0, 0)),
        ),
        scratch_shapes=[pltpu.VMEM((n_gnn, T_BLK, 4 * H), f32)],
        compiler_params=pltpu.CompilerParams(
            dimension_semantics=("arbitrary",)),
    )(x_seq, a_seq, feat_seq, w1, b1, w2, b2, fcw_flat, fcb,
      wgi, wgf, wgo, wgg, bi, bf, bo, bg,
      wii, wif, wio, wig, whi, whf, who, whg)

    return out, h_T, c_T


# speedup vs baseline: 2.7124x; 1.3754x over previous
import functools

import jax
import jax.numpy as jnp
from jax.experimental import pallas as pl
from jax.experimental.pallas import tpu as pltpu


def _dot(a, b, dims):
    return jax.lax.dot_general(a, b, (dims, ((), ())),
                               preferred_element_type=jnp.float32)


def _fused_kernel(xt_ref, a_ref, feat_ref,
                  w1t_ref, b1_ref, w2_ref, b2_ref, fcwt_ref, fcb_ref,
                  wgi_ref, wgf_ref, wgo_ref, wgg_ref,
                  bi_ref, bf_ref, bo_ref, bg_ref,
                  wii_ref, wif_ref, wio_ref, wig_ref,
                  whi_ref, whf_ref, who_ref, whg_ref,
                  out_ref, c_ref, hN_ref,
                  *, n_nodes, gh, seq_len, batch, hidden, in_dim):
    f32 = jnp.float32
    N, GH, T, B, H = n_nodes, gh, seq_len, batch, hidden
    NG = N * GH

    feat = feat_ref[...].reshape(T * N, feat_ref.shape[2])
    xw1 = _dot(feat, w1t_ref[...], ((1,), (1,)))

    col = jax.lax.broadcasted_iota(jnp.int32, (N, NG), 1)
    row = jax.lax.broadcasted_iota(jnp.int32, (N, NG), 0)
    mask = (col // GH == row).astype(f32)
    kcol = jax.lax.broadcasted_iota(jnp.int32, (GH, NG), 1)
    krow = jax.lax.broadcasted_iota(jnp.int32, (GH, NG), 0)
    ktile = (kcol % GH == krow).astype(f32)
    ones_row = jnp.ones((1, N), f32)

    h1s = []
    for t in range(T):
        h1s.append(jnp.maximum(
            _dot(a_ref[t], xw1[t * N:(t + 1) * N, :], ((1,), (0,)))
            + b1_ref[...], 0.0))
    h1 = jnp.concatenate(h1s, axis=0)
    hw = _dot(h1, w2_ref[...], ((1,), (0,)))

    flats = []
    for t in range(T):
        h2_t = (_dot(a_ref[t], hw[t * N:(t + 1) * N, :], ((1,), (0,)))
                + b2_ref[...])
        tiled = _dot(h2_t, ktile, ((1,), (0,))) * mask
        flats.append(_dot(ones_row, tiled, ((1,), (0,))))
    flat = jnp.concatenate(flats, axis=0)

    gnn = jnp.maximum(_dot(flat, fcwt_ref[...], ((1,), (1,)))
                      + fcb_ref[...], 0.0)
    wg = jnp.concatenate([wgi_ref[...], wgf_ref[...],
                          wgo_ref[...], wgg_ref[...]], axis=1)
    b4 = jnp.concatenate([bi_ref[...], bf_ref[...],
                          bo_ref[...], bg_ref[...]], axis=1)
    zg = _dot(gnn, wg, ((1,), (0,))) + b4

    wx = jnp.concatenate([wii_ref[...], wif_ref[...],
                          wio_ref[...], wig_ref[...]], axis=1)
    wh = jnp.concatenate([whi_ref[...], whf_ref[...],
                          who_ref[...], whg_ref[...]], axis=1)
    zxs = [_dot(xt_ref[t], wx, ((0,), (0,))) for t in range(T)]

    h = jnp.zeros((B, H), f32)
    c = jnp.zeros((B, H), f32)
    outs = []
    for t in range(T):
        z = zxs[t] + zg[t:t + 1, :] + _dot(h, wh, ((1,), (0,)))
        s = jax.nn.sigmoid(z[:, :3 * H])
        g_t = jnp.tanh(z[:, 3 * H:])
        c = s[:, H:2 * H] * c + s[:, :H] * g_t
        h = s[:, 2 * H:] * jnp.tanh(c)
        outs.append(h)
    out_ref[...] = jnp.stack(outs, axis=0)
    c_ref[...] = c
    hN_ref[...] = h


def kernel(x_seq, a_seq, feat_seq, w1, b1, w2, b2, fcw_flat, fcb,
           wii, wgi, whi, bi, wif, wgf, whf, bf,
           wig, wgg, whg, bg, wio, wgo, who, bo):
    T, B, I = x_seq.shape
    _, N, F = feat_seq.shape
    GH = w2.shape[0]
    H = whi.shape[0]
    f32 = jnp.float32

    xt = jnp.transpose(x_seq, (0, 2, 1))
    w1t = jnp.transpose(w1)
    fcwt = jnp.transpose(fcw_flat)

    def full(arr):
        return pl.BlockSpec(arr.shape, lambda j, _nd=arr.ndim: (0,) * _nd)

    out, c_T, h_T = pl.pallas_call(
        functools.partial(_fused_kernel, n_nodes=N, gh=GH, seq_len=T,
                          batch=B, hidden=H, in_dim=I),
        out_shape=(
            jax.ShapeDtypeStruct((T, B, H), f32),
            jax.ShapeDtypeStruct((B, H), f32),
            jax.ShapeDtypeStruct((B, H), f32),
        ),
        grid=(1,),
        in_specs=[
            full(xt), full(a_seq), full(feat_seq),
            full(w1t), full(b1), full(w2), full(b2), full(fcwt), full(fcb),
            full(wgi), full(wgf), full(wgo), full(wgg),
            full(bi), full(bf), full(bo), full(bg),
            full(wii), full(wif), full(wio), full(wig),
            full(whi), full(whf), full(who), full(whg),
        ],
        out_specs=(
            pl.BlockSpec((T, B, H), lambda j: (0, 0, 0)),
            pl.BlockSpec((B, H), lambda j: (0, 0)),
            pl.BlockSpec((B, H), lambda j: (0, 0)),
        ),
        compiler_params=pltpu.CompilerParams(
            dimension_semantics=("arbitrary",)),
    )(xt, a_seq, feat_seq, w1t, b1, w2, b2, fcwt, fcb,
      wgi, wgf, wgo, wgg, bi, bf, bo, bg,
      wii, wif, wio, wig, whi, whf, who, whg)

    return out, h_T, c_T


# speedup vs baseline: 2.7973x; 1.0313x over previous
import functools

import jax
import jax.numpy as jnp
from jax.experimental import pallas as pl
from jax.experimental.pallas import tpu as pltpu


def _dot(a, b, dims):
    return jax.lax.dot_general(a, b, (dims, ((), ())),
                               preferred_element_type=jnp.float32)


def _fused_kernel(xt_ref, a_ref, feat_ref,
                  w1t_ref, b1_ref, w2_ref, b2_ref, fcwt_ref, fcb_ref,
                  wgi_ref, wgf_ref, wgo_ref, wgg_ref,
                  bi_ref, bf_ref, bo_ref, bg_ref,
                  wii_ref, wif_ref, wio_ref, wig_ref,
                  whi_ref, whf_ref, who_ref, whg_ref,
                  out_ref, c_ref, hN_ref,
                  feat_buf, feat_sem,
                  *, n_nodes, gh, seq_len, batch, hidden, in_dim):
    f32 = jnp.float32
    N, GH, T, B, H = n_nodes, gh, seq_len, batch, hidden
    NG = N * GH

    def feat_start(slot, t):
        pltpu.make_async_copy(feat_ref.at[t], feat_buf.at[slot],
                              feat_sem.at[slot]).start()

    def feat_wait(slot):
        pltpu.make_async_copy(feat_buf.at[slot], feat_buf.at[slot],
                              feat_sem.at[slot]).wait()

    feat_start(0, 0)
    feat_start(1, 1)

    col = jax.lax.broadcasted_iota(jnp.int32, (N, NG), 1)
    row = jax.lax.broadcasted_iota(jnp.int32, (N, NG), 0)
    mask = (col // GH == row).astype(f32)
    kcol = jax.lax.broadcasted_iota(jnp.int32, (GH, NG), 1)
    krow = jax.lax.broadcasted_iota(jnp.int32, (GH, NG), 0)
    ktile = (kcol % GH == krow).astype(f32)
    ones_row = jnp.ones((1, N), f32)

    h1s = []
    for t in range(T):
        feat_wait(t % 2)
        xw1_t = _dot(feat_buf[t % 2], w1t_ref[...], ((1,), (1,)))
        if t + 2 < T:
            feat_start(t % 2, t + 2)
        h1s.append(jnp.maximum(
            _dot(a_ref[t], xw1_t, ((1,), (0,))) + b1_ref[...], 0.0))
    h1 = jnp.concatenate(h1s, axis=0)
    hw = _dot(h1, w2_ref[...], ((1,), (0,)))

    flats = []
    for t in range(T):
        h2_t = (_dot(a_ref[t], hw[t * N:(t + 1) * N, :], ((1,), (0,)))
                + b2_ref[...])
        tiled = _dot(h2_t, ktile, ((1,), (0,))) * mask
        flats.append(_dot(ones_row, tiled, ((1,), (0,))))
    flat = jnp.concatenate(flats, axis=0)

    gnn = jnp.maximum(_dot(flat, fcwt_ref[...], ((1,), (1,)))
                      + fcb_ref[...], 0.0)
    wg = jnp.concatenate([wgi_ref[...], wgf_ref[...],
                          wgo_ref[...], wgg_ref[...]], axis=1)
    b4 = jnp.concatenate([bi_ref[...], bf_ref[...],
                          bo_ref[...], bg_ref[...]], axis=1)
    zg = _dot(gnn, wg, ((1,), (0,))) + b4

    wx = jnp.concatenate([wii_ref[...], wif_ref[...],
                          wio_ref[...], wig_ref[...]], axis=1)
    wh = jnp.concatenate([whi_ref[...], whf_ref[...],
                          who_ref[...], whg_ref[...]], axis=1)
    zxs = [_dot(xt_ref[t], wx, ((0,), (0,))) for t in range(T)]

    h = jnp.zeros((B, H), f32)
    c = jnp.zeros((B, H), f32)
    outs = []
    for t in range(T):
        z = zxs[t] + zg[t:t + 1, :] + _dot(h, wh, ((1,), (0,)))
        s = jax.nn.sigmoid(z[:, :3 * H])
        g_t = jnp.tanh(z[:, 3 * H:])
        c = s[:, H:2 * H] * c + s[:, :H] * g_t
        h = s[:, 2 * H:] * jnp.tanh(c)
        outs.append(h)
    out_ref[...] = jnp.stack(outs, axis=0)
    c_ref[...] = c
    hN_ref[...] = h


def kernel(x_seq, a_seq, feat_seq, w1, b1, w2, b2, fcw_flat, fcb,
           wii, wgi, whi, bi, wif, wgf, whf, bf,
           wig, wgg, whg, bg, wio, wgo, who, bo):
    T, B, I = x_seq.shape
    _, N, F = feat_seq.shape
    GH = w2.shape[0]
    H = whi.shape[0]
    f32 = jnp.float32

    xt = jnp.transpose(x_seq, (0, 2, 1))
    w1t = jnp.transpose(w1)
    fcwt = jnp.transpose(fcw_flat)

    def full(arr):
        return pl.BlockSpec(arr.shape, lambda j, _nd=arr.ndim: (0,) * _nd)

    out, c_T, h_T = pl.pallas_call(
        functools.partial(_fused_kernel, n_nodes=N, gh=GH, seq_len=T,
                          batch=B, hidden=H, in_dim=I),
        out_shape=(
            jax.ShapeDtypeStruct((T, B, H), f32),
            jax.ShapeDtypeStruct((B, H), f32),
            jax.ShapeDtypeStruct((B, H), f32),
        ),
        grid=(1,),
        in_specs=[
            full(xt), full(a_seq),
            pl.BlockSpec(memory_space=pltpu.MemorySpace.HBM),
            full(w1t), full(b1), full(w2), full(b2), full(fcwt), full(fcb),
            full(wgi), full(wgf), full(wgo), full(wgg),
            full(bi), full(bf), full(bo), full(bg),
            full(wii), full(wif), full(wio), full(wig),
            full(whi), full(whf), full(who), full(whg),
        ],
        out_specs=(
            pl.BlockSpec((T, B, H), lambda j: (0, 0, 0)),
            pl.BlockSpec((B, H), lambda j: (0, 0)),
            pl.BlockSpec((B, H), lambda j: (0, 0)),
        ),
        scratch_shapes=[
            pltpu.VMEM((2, N, F), f32),
            pltpu.SemaphoreType.DMA((2,)),
        ],
        compiler_params=pltpu.CompilerParams(
            dimension_semantics=("arbitrary",)),
    )(xt, a_seq, feat_seq, w1t, b1, w2, b2, fcwt, fcb,
      wgi, wgf, wgo, wgg, bi, bf, bo, bg,
      wii, wif, wio, wig, whi, whf, who, whg)

    return out, h_T, c_T
